# baseline (speedup 1.0000x reference)
import functools
import jax
import jax.numpy as jnp
from jax import lax
from jax.experimental import pallas as pl
from jax.experimental.pallas import tpu as pltpu
from jax.experimental.pallas import tpu_sc as plsc

N = 8192
R = 32
NQ = 128
VOXEL = 0.2
PCS = -50.0
GAMMA = 1.1
R1, R2 = 0.8, 1.6
NS1, NS2 = 16, 32
L = 16
NCH = N // L

_mesh = plsc.VectorSubcoreMesh(core_axis_name="c", subcore_axis_name="s")
_sc_params = pltpu.CompilerParams(needs_layout_passes=False)


def _iota():
    return lax.broadcasted_iota(jnp.int32, (L,), 0)


@functools.partial(
    pl.kernel, mesh=_mesh, compiler_params=_sc_params,
    out_type=(
        jax.ShapeDtypeStruct((R * NQ * 5,), jnp.float32),
        jax.ShapeDtypeStruct((R * NQ,), jnp.float32),
        jax.ShapeDtypeStruct((R * NQ * NS1 * 5,), jnp.float32),
        jax.ShapeDtypeStruct((R * NQ * NS2 * 5,), jnp.float32),
    ),
    scratch_types=[
        pltpu.VMEM((N,), jnp.float32),
        pltpu.VMEM((N,), jnp.float32),
        pltpu.VMEM((N,), jnp.float32),
        pltpu.VMEM((N,), jnp.float32),
        pltpu.VMEM((N,), jnp.float32),
        pltpu.VMEM((N,), jnp.float32),
        pltpu.VMEM((6 * R,), jnp.float32),
        pltpu.VMEM((320,), jnp.int32),
        pltpu.VMEM((NQ * 5,), jnp.float32),
        pltpu.VMEM((NQ,), jnp.float32),
        pltpu.VMEM((NQ * NS1 * 5,), jnp.float32),
        pltpu.VMEM((NQ * NS2 * 5,), jnp.float32),
        pltpu.VMEM_SHARED((N,), jnp.float32),
    ],
)
def _sampler(px, py, pz, pi, pt, rsc, qp, qmask, g1, g2,
             pxv, pyv, pzv, piv, ptv, vmv, rscv, selv, qpl, qml,
             g1v, g2v, vsh):
    cid = lax.axis_index("c")
    sid = lax.axis_index("s")
    wid = sid * 2 + cid
    pltpu.sync_copy(px, pxv)
    pltpu.sync_copy(py, pyv)
    pltpu.sync_copy(pz, pzv)
    pltpu.sync_copy(pi, piv)
    pltpu.sync_copy(pt, ptv)
    pltpu.sync_copy(rsc, rscv)
    it = _iota()

    def vox_chunk(j, _):
        off = sid * 512 + j * L
        x = pxv[pl.ds(off, L)]
        y = pyv[pl.ds(off, L)]
        cxp = ((x - PCS) / VOXEL).astype(jnp.int32).astype(jnp.float32)
        cyp = ((y - PCS) / VOXEL).astype(jnp.int32).astype(jnp.float32)

        def roi_body(r, acc):
            qcx = plsc.load_gather(rscv, [jnp.full((L,), r, jnp.int32)])
            qcy = plsc.load_gather(rscv, [jnp.full((L,), R + r, jnp.int32)])
            rad = plsc.load_gather(rscv, [jnp.full((L,), 2 * R + r, jnp.int32)])
            hit = (jnp.abs(qcx - cxp) < rad) & (jnp.abs(qcy - cyp) < rad)
            return acc | hit

        vm = lax.fori_loop(0, R, roi_body, jnp.zeros((L,), jnp.bool_))
        vmv[pl.ds(off, L)] = jnp.where(vm, 1.0, 0.0).astype(jnp.float32)
        return 0

    lax.fori_loop(0, 512 // L, vox_chunk, 0)
    pltpu.sync_copy(vmv.at[pl.ds(sid * 512, 512)], vsh.at[pl.ds(sid * 512, 512)])
    plsc.subcore_barrier()
    pltpu.sync_copy(vsh, vmv)

    def mask_chunk(j, _):
        off = j * L
        vm = vmv[pl.ds(off, L)]
        pxv[pl.ds(off, L)] = pxv[pl.ds(off, L)] * vm
        pyv[pl.ds(off, L)] = pyv[pl.ds(off, L)] * vm
        pzv[pl.ds(off, L)] = pzv[pl.ds(off, L)] * vm
        piv[pl.ds(off, L)] = piv[pl.ds(off, L)] * vm
        ptv[pl.ds(off, L)] = ptv[pl.ds(off, L)] * vm
        return 0

    lax.fori_loop(0, NCH, mask_chunk, 0)

    cx = plsc.load_gather(rscv, [jnp.full((L,), 3 * R + wid, jnp.int32)])
    cy = plsc.load_gather(rscv, [jnp.full((L,), 4 * R + wid, jnp.int32)])
    csq = plsc.load_gather(rscv, [jnp.full((L,), 5 * R + wid, jnp.int32)])

    def pm_chunk(j, cnts):
        tc, fc = cnts
        off = j * L
        kx = pxv[pl.ds(off, L)]
        ky = pyv[pl.ds(off, L)]
        vm = vmv[pl.ds(off, L)] > 0.0
        dx = kx - cx
        dy = ky - cy
        d2 = dx * dx + dy * dy
        pm = ((d2 + 1e-12) <= csq) & vm
        pmi = jnp.where(pm, 1, 0).astype(jnp.int32)
        nt = jnp.sum(pmi)
        idxc = off + it

        @pl.when((tc < NQ) & (nt > 0))
        def _():
            slots = tc + plsc.cumsum(pmi) - pmi
            plsc.store_scatter(selv, [slots], idxc, mask=pm & (slots < 160))

        fmi = 1 - pmi
        nf = jnp.sum(fmi)

        @pl.when(fc < NQ)
        def _():
            fslots = 160 + fc + plsc.cumsum(fmi) - fmi
            plsc.store_scatter(selv, [fslots], idxc,
                               mask=(~pm) & (fslots < 320))

        return tc + nt, fc + nf

    tc, _fc = lax.fori_loop(0, NCH, pm_chunk,
                            (jnp.int32(0), jnp.int32(0)))

    def qgather(j, _):
        lane = j * L + it
        sel = lane < tc
        gidx = jnp.where(sel, lane, 160 + lane - tc)
        midx = plsc.load_gather(selv, [gidx])
        rows = lane
        qm = jnp.where(sel, 1.0, 0.0).astype(jnp.float32)
        qml[pl.ds(j * L, L)] = qm
        for ch, plane in enumerate((pxv, pyv, pzv, piv, ptv)):
            v = plsc.load_gather(plane, [midx])
            plsc.store_scatter(qpl, [rows * 5 + ch], v)
        return 0

    lax.fori_loop(0, NQ // L, qgather, 0)
    pltpu.sync_copy(qpl, qp.at[pl.ds(wid * NQ * 5, NQ * 5)])
    pltpu.sync_copy(qml, qmask.at[pl.ds(wid * NQ, NQ)])

    def zero1(j, _):
        g1v[pl.ds(j * L, L)] = jnp.zeros((L,), jnp.float32)
        return 0

    def zero2(j, _):
        g2v[pl.ds(j * L, L)] = jnp.zeros((L,), jnp.float32)
        return 0

    lax.fori_loop(0, NQ * NS1 * 5 // L, zero1, 0)
    lax.fori_loop(0, NQ * NS2 * 5 // L, zero2, 0)

    def compact(j, kn):
        off = j * L
        m = vmv[pl.ds(off, L)] > 0.0
        mi = jnp.where(m, 1, 0).astype(jnp.int32)
        nt = jnp.sum(mi)

        @pl.when(nt > 0)
        def _():
            slots = kn + plsc.cumsum(mi) - mi
            for plane in (pxv, pyv, pzv, piv, ptv):
                v = plane[pl.ds(off, L)]
                plsc.store_scatter(plane, [slots], v, mask=m)

        return kn + nt

    kn = lax.fori_loop(0, NCH, compact, jnp.int32(0))
    big = jnp.full((L,), 1e30, jnp.float32)
    tails = kn + it
    tmask = tails < N
    plsc.store_scatter(pxv, [tails], big, mask=tmask)
    plsc.store_scatter(pyv, [tails], big, mask=tmask)
    plsc.store_scatter(pzv, [tails], big, mask=tmask)
    nch = (kn + (L - 1)) // L
    r1sq = jnp.float32(R1 * R1)
    r2sq = jnp.float32(R2 * R2)

    def per_query(ql, _):
        qx = plsc.load_gather(qpl, [jnp.full((L,), ql * 5 + 0, jnp.int32)])
        qy = plsc.load_gather(qpl, [jnp.full((L,), ql * 5 + 1, jnp.int32)])
        qz = plsc.load_gather(qpl, [jnp.full((L,), ql * 5 + 2, jnp.int32)])
        tzv = qx * qx + qy * qy + qz * qz
        tz = tzv[0]
        tzs = tzv * jnp.float32(0.999999)
        t1 = jnp.minimum(r1sq, tzs)
        t2 = jnp.minimum(r2sq, tzs)

        def scan_chunk(j, cnts):
            off = j * L
            kx = pxv[pl.ds(off, L)]
            ky = pyv[pl.ds(off, L)]
            kz = pzv[pl.ds(off, L)]
            dx = kx - qx
            dy = ky - qy
            dz = kz - qz
            d2 = dx * dx + dy * dy + dz * dz
            m2 = d2 < t2
            m2i = jnp.where(m2, 1, 0).astype(jnp.int32)
            n2 = jnp.sum(m2i)

            def hit(c):
                c1, c2 = c
                m1 = d2 < t1
                m1i = jnp.where(m1, 1, 0).astype(jnp.int32)
                n1 = jnp.sum(m1i)
                ki = piv[pl.ds(off, L)]
                kt = ptv[pl.ds(off, L)]
                s1 = c1 + plsc.cumsum(m1i) - m1i
                ok1 = m1 & (s1 < NS1)
                b1 = (ql * NS1 + s1) * 5
                plsc.store_scatter(g1v, [b1], dx, mask=ok1)
                plsc.store_scatter(g1v, [b1 + 1], dy, mask=ok1)
                plsc.store_scatter(g1v, [b1 + 2], dz, mask=ok1)
                plsc.store_scatter(g1v, [b1 + 3], ki, mask=ok1)
                plsc.store_scatter(g1v, [b1 + 4], kt, mask=ok1)
                s2 = c2 + plsc.cumsum(m2i) - m2i
                ok2 = m2 & (s2 < NS2)
                b2 = (ql * NS2 + s2) * 5
                plsc.store_scatter(g2v, [b2], dx, mask=ok2)
                plsc.store_scatter(g2v, [b2 + 1], dy, mask=ok2)
                plsc.store_scatter(g2v, [b2 + 2], dz, mask=ok2)
                plsc.store_scatter(g2v, [b2 + 3], ki, mask=ok2)
                plsc.store_scatter(g2v, [b2 + 4], kt, mask=ok2)
                return c1 + n1, c2 + n2

            return lax.cond(n2 > 0, hit, lambda c: c, cnts)

        c1, c2 = lax.fori_loop(0, nch, scan_chunk,
                               (jnp.int32(0), jnp.int32(0)))

        vals = jnp.where(it == 0, -qx,
                         jnp.where(it == 1, -qy,
                                   jnp.where(it == 2, -qz, 0.0)))
        vals = vals.astype(jnp.float32)

        @pl.when((tz <= r1sq) & (c1 < NS1))
        def _():
            plsc.store_scatter(g1v, [(ql * NS1 + c1) * 5 + it], vals,
                               mask=it < 5)

        @pl.when((tz <= r2sq) & (c2 < NS2))
        def _():
            plsc.store_scatter(g2v, [(ql * NS2 + c2) * 5 + it], vals,
                               mask=it < 5)

        return 0

    lax.fori_loop(0, NQ, per_query, 0)
    pltpu.sync_copy(g1v, g1.at[pl.ds(wid * NQ * NS1 * 5, NQ * NS1 * 5)])
    pltpu.sync_copy(g2v, g2.at[pl.ds(wid * NQ * NS2 * 5, NQ * NS2 * 5)])


def _mlp_body(g1r, g2r, w11, w12, w13, b11, b12, b13,
              w21, w22, w23, b21, b22, b23, o1r, o2r):
    def mlp(g, ws, bs):
        h = g
        for w, b in zip(ws, bs):
            h = jnp.maximum(
                jnp.dot(h, w[...], preferred_element_type=jnp.float32)
                + b[...], 0.0)
        return h

    h1 = mlp(g1r[...], (w11, w12, w13), (b11, b12, b13))
    nq = o1r.shape[0]
    o1r[...] = jnp.max(h1.reshape(nq, NS1, h1.shape[-1]), axis=1)
    h2 = mlp(g2r[...], (w21, w22, w23), (b21, b22, b23))
    o2r[...] = jnp.max(h2.reshape(nq, NS2, h2.shape[-1]), axis=1)


def _mlp_tc(g1, g2, w11, w12, w13, b11, b12, b13,
            w21, w22, w23, b21, b22, b23):
    nblk = 8
    qb = R * NQ // nblk
    wspec = lambda a: pl.BlockSpec(a.shape, lambda i: (0,) * a.ndim)
    return pl.pallas_call(
        _mlp_body,
        grid=(nblk,),
        in_specs=[
            pl.BlockSpec((qb * NS1, 5), lambda i: (i, 0)),
            pl.BlockSpec((qb * NS2, 5), lambda i: (i, 0)),
            wspec(w11), wspec(w12), wspec(w13),
            wspec(b11), wspec(b12), wspec(b13),
            wspec(w21), wspec(w22), wspec(w23),
            wspec(b21), wspec(b22), wspec(b23),
        ],
        out_specs=[
            pl.BlockSpec((qb, 32), lambda i: (i, 0)),
            pl.BlockSpec((qb, 64), lambda i: (i, 0)),
        ],
        out_shape=[
            jax.ShapeDtypeStruct((R * NQ, 32), jnp.float32),
            jax.ShapeDtypeStruct((R * NQ, 64), jnp.float32),
        ],
    )(g1, g2, w11, w12, w13, b11, b12, b13,
      w21, w22, w23, b21, b22, b23)


def kernel(points, trajectory_rois, b1_w1, b1_w2, b1_w3, b1_b1, b1_b2, b1_b3,
           b2_w1, b2_w2, b2_w3, b2_b1, b2_b2, b2_b3):
    rois = trajectory_rois[0, 0]
    half = rois[:, 3:5] / 2.0
    nrm = jnp.sqrt(jnp.sum(half * half, axis=-1))
    qc = jnp.floor((rois[:, :2] - jnp.float32(PCS)) / VOXEL)
    rad = jnp.ceil(nrm * GAMMA / VOXEL)
    cur = nrm * GAMMA
    rsc = jnp.stack([qc[:, 0], qc[:, 1], rad, rois[:, 0], rois[:, 1],
                     cur * cur]).astype(jnp.float32).reshape(-1)
    p32 = points.astype(jnp.float32)
    qpf, qmask, g1f, g2f = _sampler(p32[:, 0], p32[:, 1], p32[:, 2],
                                    p32[:, 3], p32[:, 4], rsc)
    qp = qpf.reshape(R * NQ, 5)
    f1, f2 = _mlp_tc(g1f.reshape(R * NQ * NS1, 5), g2f.reshape(R * NQ * NS2, 5),
                     b1_w1, b1_w2, b1_w3, b1_b1, b1_b2, b1_b3,
                     b2_w1, b2_w2, b2_w3, b2_b1, b2_b2, b2_b3)
    pf = jnp.concatenate([qp[:, :3], f1, f2], axis=-1)
    sp = qp * qmask[:, None]
    return (sp.reshape(1, R, NQ, 5),
            pf.reshape(1, R, NQ, 3 + 32 + 64))


# speedup vs baseline: 1.0438x; 1.0438x over previous
import functools
import jax
import jax.numpy as jnp
from jax import lax
from jax.experimental import pallas as pl
from jax.experimental.pallas import tpu as pltpu
from jax.experimental.pallas import tpu_sc as plsc

N = 8192
R = 32
NQ = 128
VOXEL = 0.2
PCS = -50.0
GAMMA = 1.1
R1, R2 = 0.8, 1.6
NS1, NS2 = 16, 32
L = 16
NCH = N // L

_mesh = plsc.VectorSubcoreMesh(core_axis_name="c", subcore_axis_name="s")
_sc_params = pltpu.CompilerParams(needs_layout_passes=False)


def _iota():
    return lax.broadcasted_iota(jnp.int32, (L,), 0)


@functools.partial(
    pl.kernel, mesh=_mesh, compiler_params=_sc_params,
    out_type=(
        jax.ShapeDtypeStruct((R * NQ * 5,), jnp.float32),
        jax.ShapeDtypeStruct((R * NQ,), jnp.float32),
        jax.ShapeDtypeStruct((R * NQ * NS1 * 5,), jnp.float32),
        jax.ShapeDtypeStruct((R * NQ * NS2 * 5,), jnp.float32),
    ),
    scratch_types=[
        pltpu.VMEM((N,), jnp.float32),
        pltpu.VMEM((N,), jnp.float32),
        pltpu.VMEM((N,), jnp.float32),
        pltpu.VMEM((N,), jnp.float32),
        pltpu.VMEM((N,), jnp.float32),
        pltpu.VMEM((N,), jnp.float32),
        pltpu.VMEM((6 * R,), jnp.float32),
        pltpu.VMEM((320,), jnp.int32),
        pltpu.VMEM((NQ * 5,), jnp.float32),
        pltpu.VMEM((NQ,), jnp.float32),
        pltpu.VMEM((NQ * NS1 * 5,), jnp.float32),
        pltpu.VMEM((NQ * NS2 * 5,), jnp.float32),
        pltpu.VMEM_SHARED((N,), jnp.float32),
    ],
)
def _sampler(px, py, pz, pi, pt, rsc, qp, qmask, g1, g2,
             pxv, pyv, pzv, piv, ptv, vmv, rscv, selv, qpl, qml,
             g1v, g2v, vsh):
    cid = lax.axis_index("c")
    sid = lax.axis_index("s")
    wid = sid * 2 + cid
    pltpu.sync_copy(px, pxv)
    pltpu.sync_copy(py, pyv)
    pltpu.sync_copy(pz, pzv)
    pltpu.sync_copy(pi, piv)
    pltpu.sync_copy(pt, ptv)
    pltpu.sync_copy(rsc, rscv)
    it = _iota()

    def vox_chunk(j, _):
        off = sid * 512 + j * L
        x = pxv[pl.ds(off, L)]
        y = pyv[pl.ds(off, L)]
        cxp = ((x - PCS) / VOXEL).astype(jnp.int32).astype(jnp.float32)
        cyp = ((y - PCS) / VOXEL).astype(jnp.int32).astype(jnp.float32)

        def roi_body(r, acc):
            qcx = plsc.load_gather(rscv, [jnp.full((L,), r, jnp.int32)])
            qcy = plsc.load_gather(rscv, [jnp.full((L,), R + r, jnp.int32)])
            rad = plsc.load_gather(rscv, [jnp.full((L,), 2 * R + r, jnp.int32)])
            hit = (jnp.abs(qcx - cxp) < rad) & (jnp.abs(qcy - cyp) < rad)
            return acc | hit

        vm = lax.fori_loop(0, R, roi_body, jnp.zeros((L,), jnp.bool_))
        vmv[pl.ds(off, L)] = jnp.where(vm, 1.0, 0.0).astype(jnp.float32)
        return 0

    lax.fori_loop(0, 512 // L, vox_chunk, 0)
    pltpu.sync_copy(vmv.at[pl.ds(sid * 512, 512)], vsh.at[pl.ds(sid * 512, 512)])
    plsc.subcore_barrier()
    pltpu.sync_copy(vsh, vmv)

    def mask_chunk(j, _):
        off = j * L
        vm = vmv[pl.ds(off, L)]
        pxv[pl.ds(off, L)] = pxv[pl.ds(off, L)] * vm
        pyv[pl.ds(off, L)] = pyv[pl.ds(off, L)] * vm
        pzv[pl.ds(off, L)] = pzv[pl.ds(off, L)] * vm
        piv[pl.ds(off, L)] = piv[pl.ds(off, L)] * vm
        ptv[pl.ds(off, L)] = ptv[pl.ds(off, L)] * vm
        return 0

    lax.fori_loop(0, NCH, mask_chunk, 0)

    cx = plsc.load_gather(rscv, [jnp.full((L,), 3 * R + wid, jnp.int32)])
    cy = plsc.load_gather(rscv, [jnp.full((L,), 4 * R + wid, jnp.int32)])
    csq = plsc.load_gather(rscv, [jnp.full((L,), 5 * R + wid, jnp.int32)])

    def pm_chunk(j, cnts):
        tc, fc = cnts
        off = j * L
        kx = pxv[pl.ds(off, L)]
        ky = pyv[pl.ds(off, L)]
        vm = vmv[pl.ds(off, L)] > 0.0
        dx = kx - cx
        dy = ky - cy
        d2 = dx * dx + dy * dy
        pm = ((d2 + 1e-12) <= csq) & vm
        pmi = jnp.where(pm, 1, 0).astype(jnp.int32)
        nt = jnp.sum(pmi)
        idxc = off + it

        @pl.when((tc < NQ) & (nt > 0))
        def _():
            slots = tc + plsc.cumsum(pmi) - pmi
            plsc.store_scatter(selv, [slots], idxc, mask=pm & (slots < 160))

        fmi = 1 - pmi
        nf = jnp.sum(fmi)

        @pl.when(fc < NQ)
        def _():
            fslots = 160 + fc + plsc.cumsum(fmi) - fmi
            plsc.store_scatter(selv, [fslots], idxc,
                               mask=(~pm) & (fslots < 320))

        return tc + nt, fc + nf

    tc, _fc = lax.fori_loop(0, NCH, pm_chunk,
                            (jnp.int32(0), jnp.int32(0)))

    def qgather(j, _):
        lane = j * L + it
        sel = lane < tc
        gidx = jnp.where(sel, lane, 160 + lane - tc)
        midx = plsc.load_gather(selv, [gidx])
        rows = lane
        qm = jnp.where(sel, 1.0, 0.0).astype(jnp.float32)
        qml[pl.ds(j * L, L)] = qm
        for ch, plane in enumerate((pxv, pyv, pzv, piv, ptv)):
            v = plsc.load_gather(plane, [midx])
            plsc.store_scatter(qpl, [rows * 5 + ch], v)
        return 0

    lax.fori_loop(0, NQ // L, qgather, 0)
    pltpu.sync_copy(qpl, qp.at[pl.ds(wid * NQ * 5, NQ * 5)])
    pltpu.sync_copy(qml, qmask.at[pl.ds(wid * NQ, NQ)])

    def zero1(j, _):
        g1v[pl.ds(j * L, L)] = jnp.zeros((L,), jnp.float32)
        return 0

    def zero2(j, _):
        g2v[pl.ds(j * L, L)] = jnp.zeros((L,), jnp.float32)
        return 0

    lax.fori_loop(0, NQ * NS1 * 5 // L, zero1, 0)
    lax.fori_loop(0, NQ * NS2 * 5 // L, zero2, 0)

    def compact(j, kn):
        off = j * L
        m = vmv[pl.ds(off, L)] > 0.0
        mi = jnp.where(m, 1, 0).astype(jnp.int32)
        nt = jnp.sum(mi)

        @pl.when(nt > 0)
        def _():
            slots = kn + plsc.cumsum(mi) - mi
            for plane in (pxv, pyv, pzv, piv, ptv):
                v = plane[pl.ds(off, L)]
                plsc.store_scatter(plane, [slots], v, mask=m)

        return kn + nt

    kn = lax.fori_loop(0, NCH, compact, jnp.int32(0))
    big = jnp.full((L,), 1e30, jnp.float32)
    tails = kn + it
    tmask = tails < N
    plsc.store_scatter(pxv, [tails], big, mask=tmask)
    plsc.store_scatter(pyv, [tails], big, mask=tmask)
    plsc.store_scatter(pzv, [tails], big, mask=tmask)
    nch = (kn + (L - 1)) // L
    r1sq = jnp.float32(R1 * R1)
    r2sq = jnp.float32(R2 * R2)

    def per_query(ql, _):
        qx = plsc.load_gather(qpl, [jnp.full((L,), ql * 5 + 0, jnp.int32)])
        qy = plsc.load_gather(qpl, [jnp.full((L,), ql * 5 + 1, jnp.int32)])
        qz = plsc.load_gather(qpl, [jnp.full((L,), ql * 5 + 2, jnp.int32)])
        tzv = qx * qx + qy * qy + qz * qz
        tz = tzv[0]
        tzs = tzv * jnp.float32(0.999999)
        t1 = jnp.minimum(r1sq, tzs)
        t2 = jnp.minimum(r2sq, tzs)

        def scan_chunk(j, cnts):
            off = j * L
            kx = pxv[pl.ds(off, L)]
            ky = pyv[pl.ds(off, L)]
            kz = pzv[pl.ds(off, L)]
            dx = kx - qx
            dy = ky - qy
            dz = kz - qz
            d2 = dx * dx + dy * dy + dz * dz
            m2 = d2 < t2
            m2i = jnp.where(m2, 1, 0).astype(jnp.int32)
            n2 = jnp.sum(m2i)
            m1 = d2 < t1
            m1i = jnp.where(m1, 1, 0).astype(jnp.int32)
            n1 = jnp.sum(m1i)
            c1, c2 = cnts

            @pl.when(n2 > 0)
            def _():
                ki = piv[pl.ds(off, L)]
                kt = ptv[pl.ds(off, L)]
                s1 = c1 + plsc.cumsum(m1i) - m1i
                ok1 = m1 & (s1 < NS1)
                b1 = (ql * NS1 + s1) * 5
                plsc.store_scatter(g1v, [b1], dx, mask=ok1)
                plsc.store_scatter(g1v, [b1 + 1], dy, mask=ok1)
                plsc.store_scatter(g1v, [b1 + 2], dz, mask=ok1)
                plsc.store_scatter(g1v, [b1 + 3], ki, mask=ok1)
                plsc.store_scatter(g1v, [b1 + 4], kt, mask=ok1)
                s2 = c2 + plsc.cumsum(m2i) - m2i
                ok2 = m2 & (s2 < NS2)
                b2 = (ql * NS2 + s2) * 5
                plsc.store_scatter(g2v, [b2], dx, mask=ok2)
                plsc.store_scatter(g2v, [b2 + 1], dy, mask=ok2)
                plsc.store_scatter(g2v, [b2 + 2], dz, mask=ok2)
                plsc.store_scatter(g2v, [b2 + 3], ki, mask=ok2)
                plsc.store_scatter(g2v, [b2 + 4], kt, mask=ok2)

            return c1 + n1, c2 + n2

        c1, c2 = lax.fori_loop(0, nch, scan_chunk,
                               (jnp.int32(0), jnp.int32(0)))

        vals = jnp.where(it == 0, -qx,
                         jnp.where(it == 1, -qy,
                                   jnp.where(it == 2, -qz, 0.0)))
        vals = vals.astype(jnp.float32)

        @pl.when((tz <= r1sq) & (c1 < NS1))
        def _():
            plsc.store_scatter(g1v, [(ql * NS1 + c1) * 5 + it], vals,
                               mask=it < 5)

        @pl.when((tz <= r2sq) & (c2 < NS2))
        def _():
            plsc.store_scatter(g2v, [(ql * NS2 + c2) * 5 + it], vals,
                               mask=it < 5)

        return 0

    lax.fori_loop(0, NQ, per_query, 0)
    pltpu.sync_copy(g1v, g1.at[pl.ds(wid * NQ * NS1 * 5, NQ * NS1 * 5)])
    pltpu.sync_copy(g2v, g2.at[pl.ds(wid * NQ * NS2 * 5, NQ * NS2 * 5)])


def _mlp_body(g1r, g2r, w11, w12, w13, b11, b12, b13,
              w21, w22, w23, b21, b22, b23, o1r, o2r):
    def mlp(g, ws, bs):
        h = g
        for w, b in zip(ws, bs):
            h = jnp.maximum(
                jnp.dot(h, w[...], preferred_element_type=jnp.float32)
                + b[...], 0.0)
        return h

    h1 = mlp(g1r[...], (w11, w12, w13), (b11, b12, b13))
    nq = o1r.shape[0]
    o1r[...] = jnp.max(h1.reshape(nq, NS1, h1.shape[-1]), axis=1)
    h2 = mlp(g2r[...], (w21, w22, w23), (b21, b22, b23))
    o2r[...] = jnp.max(h2.reshape(nq, NS2, h2.shape[-1]), axis=1)


def _mlp_tc(g1, g2, w11, w12, w13, b11, b12, b13,
            w21, w22, w23, b21, b22, b23):
    nblk = 8
    qb = R * NQ // nblk
    wspec = lambda a: pl.BlockSpec(a.shape, lambda i: (0,) * a.ndim)
    return pl.pallas_call(
        _mlp_body,
        grid=(nblk,),
        in_specs=[
            pl.BlockSpec((qb * NS1, 5), lambda i: (i, 0)),
            pl.BlockSpec((qb * NS2, 5), lambda i: (i, 0)),
            wspec(w11), wspec(w12), wspec(w13),
            wspec(b11), wspec(b12), wspec(b13),
            wspec(w21), wspec(w22), wspec(w23),
            wspec(b21), wspec(b22), wspec(b23),
        ],
        out_specs=[
            pl.BlockSpec((qb, 32), lambda i: (i, 0)),
            pl.BlockSpec((qb, 64), lambda i: (i, 0)),
        ],
        out_shape=[
            jax.ShapeDtypeStruct((R * NQ, 32), jnp.float32),
            jax.ShapeDtypeStruct((R * NQ, 64), jnp.float32),
        ],
    )(g1, g2, w11, w12, w13, b11, b12, b13,
      w21, w22, w23, b21, b22, b23)


def kernel(points, trajectory_rois, b1_w1, b1_w2, b1_w3, b1_b1, b1_b2, b1_b3,
           b2_w1, b2_w2, b2_w3, b2_b1, b2_b2, b2_b3):
    rois = trajectory_rois[0, 0]
    half = rois[:, 3:5] / 2.0
    nrm = jnp.sqrt(jnp.sum(half * half, axis=-1))
    qc = jnp.floor((rois[:, :2] - jnp.float32(PCS)) / VOXEL)
    rad = jnp.ceil(nrm * GAMMA / VOXEL)
    cur = nrm * GAMMA
    rsc = jnp.stack([qc[:, 0], qc[:, 1], rad, rois[:, 0], rois[:, 1],
                     cur * cur]).astype(jnp.float32).reshape(-1)
    p32 = points.astype(jnp.float32)
    qpf, qmask, g1f, g2f = _sampler(p32[:, 0], p32[:, 1], p32[:, 2],
                                    p32[:, 3], p32[:, 4], rsc)
    qp = qpf.reshape(R * NQ, 5)
    f1, f2 = _mlp_tc(g1f.reshape(R * NQ * NS1, 5), g2f.reshape(R * NQ * NS2, 5),
                     b1_w1, b1_w2, b1_w3, b1_b1, b1_b2, b1_b3,
                     b2_w1, b2_w2, b2_w3, b2_b1, b2_b2, b2_b3)
    pf = jnp.concatenate([qp[:, :3], f1, f2], axis=-1)
    sp = qp * qmask[:, None]
    return (sp.reshape(1, R, NQ, 5),
            pf.reshape(1, R, NQ, 3 + 32 + 64))
